# baseline (speedup 1.0000x reference)
import functools

import jax
import jax.numpy as jnp
from jax import lax
from jax.experimental import pallas as pl
from jax.experimental.pallas import tpu as pltpu
from jax.experimental.pallas import tpu_sc as plsc

_EMB = 448


def _mean_body(nblk, inv_n, x_ref, o_ref, acc_ref):
    i = pl.program_id(0)

    @pl.when(i == 0)
    def _init():
        acc_ref[...] = jnp.zeros_like(acc_ref)

    x = x_ref[...]
    c, bh, w = x.shape
    p = x[:, 0:8, :]
    for g in range(1, bh // 8):
        p = p + x[:, 8 * g : 8 * (g + 1), :]
    acc_ref[...] += p

    @pl.when(i == nblk - 1)
    def _fin():
        o_ref[...] = acc_ref[...].sum(axis=(1, 2))[:, None] * inv_n


def _channel_means_tc(image):
    c, h, w = image.shape
    bh = 64
    assert h % bh == 0
    nblk = h // bh
    return pl.pallas_call(
        functools.partial(_mean_body, nblk, 1.0 / (h * w)),
        grid=(nblk,),
        in_specs=[pl.BlockSpec((c, bh, w), lambda i: (0, i, 0))],
        out_specs=pl.BlockSpec((c, 1), lambda i: (0, 0)),
        out_shape=jax.ShapeDtypeStruct((c, 1), jnp.float32),
        scratch_shapes=[pltpu.VMEM((c, 8, w), jnp.float32)],
    )(image)


def _make_hist_sc(h, w, nbins):
    info = plsc.get_sparse_core_info()
    nc, ns, nl = info.num_cores, info.num_subcores, info.num_lanes
    nw = nc * ns
    hw = 16
    lrw = h // hw
    assert h % hw == 0 and lrw % 8 == 0 and w % nl == 0
    nvec = w // nl
    mesh = plsc.VectorSubcoreMesh(core_axis_name="c", subcore_axis_name="s")

    @functools.partial(
        pl.kernel,
        mesh=mesh,
        compiler_params=pltpu.CompilerParams(
            needs_layout_passes=False,
            use_tc_tiling_on_sc=True,
            skip_device_barrier=True,
        ),
        out_type=jax.ShapeDtypeStruct((hw * nbins,), jnp.float32),
        scratch_types=[
            pltpu.VMEM((lrw, w), jnp.int32),
            pltpu.VMEM((nl * nbins,), jnp.float32),
            pltpu.VMEM((nbins,), jnp.float32),
            pltpu.SemaphoreType.DMA,
        ],
    )
    def sc_kernel(lbl_hbm, hist_out, lbl_v, hist_v, part_v, sem_l):
        wid = lax.axis_index("s") * nc + lax.axis_index("c")

        @pl.when(wid < hw)
        def _hist():
            lbl_cp = pltpu.async_copy(
                lbl_hbm.at[pl.ds(wid * lrw, lrw)], lbl_v, sem_l
            )

            def _zero(t, carry):
                for u in range(8):
                    hist_v[pl.ds((t * 8 + u) * nl, nl)] = jnp.zeros(
                        (nl,), jnp.float32
                    )
                return carry

            lax.fori_loop(0, (nl * nbins) // (nl * 8), _zero, 0)
            lbl_cp.wait()

            lane_base = lax.iota(jnp.int32, nl) * nbins
            ones = jnp.ones((nl,), jnp.float32)

            def _scat(r, carry):
                for v in range(nvec):
                    idx = lbl_v[r, pl.ds(v * nl, nl)]
                    plsc.addupdate_scatter(hist_v, [lane_base + idx], ones)
                return carry

            lax.fori_loop(0, lrw, _scat, 0)

            def _red(cchunk, carry):
                def _lane(l, acc):
                    a0, a1 = acc
                    b = 2 * l * nbins + cchunk * nl
                    return (a0 + hist_v[pl.ds(b, nl)],
                            a1 + hist_v[pl.ds(b + nbins, nl)])

                z = jnp.zeros((nl,), jnp.float32)
                a0, a1 = lax.fori_loop(0, nl // 2, _lane, (z, z))
                part_v[pl.ds(cchunk * nl, nl)] = a0 + a1
                return carry

            lax.fori_loop(0, nbins // nl, _red, 0)

            pltpu.sync_copy(part_v, hist_out.at[pl.ds(wid * nbins, nbins)])

    return sc_kernel


def kernel(image, label):
    c, h, w = image.shape
    n = h * w
    nbins = _EMB - c
    mean_c = _channel_means_tc(image)[:, 0]
    hist_parts = _make_hist_sc(h, w, nbins)(label)
    hist = hist_parts.reshape(-1, nbins).sum(axis=0) * (1.0 / n)
    return jnp.concatenate([mean_c, hist])


# speedup vs baseline: 1.0060x; 1.0060x over previous
import functools

import jax
import jax.numpy as jnp
from jax import lax
from jax.experimental import pallas as pl
from jax.experimental.pallas import tpu as pltpu
from jax.experimental.pallas import tpu_sc as plsc

_EMB = 448


def _mean_body(nblk, inv_n, x_ref, o_ref, acc_ref):
    i = pl.program_id(0)

    @pl.when(i == 0)
    def _init():
        acc_ref[...] = jnp.zeros_like(acc_ref)

    x = x_ref[...]
    c, bh, w = x.shape
    p = x[:, 0:8, :]
    for g in range(1, bh // 8):
        p = p + x[:, 8 * g : 8 * (g + 1), :]
    acc_ref[...] += p

    @pl.when(i == nblk - 1)
    def _fin():
        o_ref[...] = acc_ref[...].sum(axis=(1, 2))[:, None] * inv_n


def _channel_means_tc(image):
    c, h, w = image.shape
    bh = 48
    assert h % bh == 0
    nblk = h // bh
    return pl.pallas_call(
        functools.partial(_mean_body, nblk, 1.0 / (h * w)),
        grid=(nblk,),
        in_specs=[pl.BlockSpec((c, bh, w), lambda i: (0, i, 0))],
        out_specs=pl.BlockSpec((c, 1), lambda i: (0, 0)),
        out_shape=jax.ShapeDtypeStruct((c, 1), jnp.float32),
        scratch_shapes=[pltpu.VMEM((c, 8, w), jnp.float32)],
    )(image)


def _make_hist_sc(h, w, nbins):
    info = plsc.get_sparse_core_info()
    nc, ns, nl = info.num_cores, info.num_subcores, info.num_lanes
    nw = nc * ns
    hw = 16
    lrw = h // hw
    assert h % hw == 0 and lrw % 8 == 0 and w % nl == 0
    nvec = w // nl
    mesh = plsc.VectorSubcoreMesh(core_axis_name="c", subcore_axis_name="s")

    @functools.partial(
        pl.kernel,
        mesh=mesh,
        compiler_params=pltpu.CompilerParams(
            needs_layout_passes=False,
            use_tc_tiling_on_sc=True,
            skip_device_barrier=True,
        ),
        out_type=jax.ShapeDtypeStruct((hw * nbins,), jnp.float32),
        scratch_types=[
            pltpu.VMEM((lrw, w), jnp.int32),
            pltpu.VMEM((nl * nbins,), jnp.float32),
            pltpu.VMEM((nbins,), jnp.float32),
            pltpu.SemaphoreType.DMA,
        ],
    )
    def sc_kernel(lbl_hbm, hist_out, lbl_v, hist_v, part_v, sem_l):
        wid = lax.axis_index("s") * nc + lax.axis_index("c")

        @pl.when(wid < hw)
        def _hist():
            lbl_cp = pltpu.async_copy(
                lbl_hbm.at[pl.ds(wid * lrw, lrw)], lbl_v, sem_l
            )

            def _zero(t, carry):
                for u in range(8):
                    hist_v[pl.ds((t * 8 + u) * nl, nl)] = jnp.zeros(
                        (nl,), jnp.float32
                    )
                return carry

            lax.fori_loop(0, (nl * nbins) // (nl * 8), _zero, 0)
            lbl_cp.wait()

            lane_base = lax.iota(jnp.int32, nl) * nbins
            ones = jnp.ones((nl,), jnp.float32)

            def _scat(r, carry):
                for v in range(nvec):
                    idx = lbl_v[r, pl.ds(v * nl, nl)]
                    plsc.addupdate_scatter(hist_v, [lane_base + idx], ones)
                return carry

            lax.fori_loop(0, lrw, _scat, 0)

            def _red(cchunk, carry):
                def _lane(l, acc):
                    a0, a1 = acc
                    b = 2 * l * nbins + cchunk * nl
                    return (a0 + hist_v[pl.ds(b, nl)],
                            a1 + hist_v[pl.ds(b + nbins, nl)])

                z = jnp.zeros((nl,), jnp.float32)
                a0, a1 = lax.fori_loop(0, nl // 2, _lane, (z, z))
                part_v[pl.ds(cchunk * nl, nl)] = a0 + a1
                return carry

            lax.fori_loop(0, nbins // nl, _red, 0)

            pltpu.sync_copy(part_v, hist_out.at[pl.ds(wid * nbins, nbins)])

    return sc_kernel


def kernel(image, label):
    c, h, w = image.shape
    n = h * w
    nbins = _EMB - c
    mean_c = _channel_means_tc(image)[:, 0]
    hist_parts = _make_hist_sc(h, w, nbins)(label)
    hist = hist_parts.reshape(-1, nbins).sum(axis=0) * (1.0 / n)
    return jnp.concatenate([mean_c, hist])


# speedup vs baseline: 1.0100x; 1.0040x over previous
import functools

import jax
import jax.numpy as jnp
from jax import lax
from jax.experimental import pallas as pl
from jax.experimental.pallas import tpu as pltpu
from jax.experimental.pallas import tpu_sc as plsc

_EMB = 448


def _mean_body(nblk, inv_n, x_ref, o_ref, acc_ref):
    i = pl.program_id(0)

    @pl.when(i == 0)
    def _init():
        acc_ref[...] = jnp.zeros_like(acc_ref)

    x = x_ref[...]
    c, bh, w = x.shape
    p = x[:, 0:8, :]
    for g in range(1, bh // 8):
        p = p + x[:, 8 * g : 8 * (g + 1), :]
    acc_ref[...] += p

    @pl.when(i == nblk - 1)
    def _fin():
        o_ref[...] = acc_ref[...].sum(axis=(1, 2))[:, None] * inv_n


def _channel_means_tc(image):
    c, h, w = image.shape
    bh = 32
    assert h % bh == 0
    nblk = h // bh
    return pl.pallas_call(
        functools.partial(_mean_body, nblk, 1.0 / (h * w)),
        grid=(nblk,),
        in_specs=[pl.BlockSpec((c, bh, w), lambda i: (0, i, 0))],
        out_specs=pl.BlockSpec((c, 1), lambda i: (0, 0)),
        out_shape=jax.ShapeDtypeStruct((c, 1), jnp.float32),
        scratch_shapes=[pltpu.VMEM((c, 8, w), jnp.float32)],
    )(image)


def _make_hist_sc(h, w, nbins):
    info = plsc.get_sparse_core_info()
    nc, ns, nl = info.num_cores, info.num_subcores, info.num_lanes
    nw = nc * ns
    hw = 16
    lrw = h // hw
    assert h % hw == 0 and lrw % 8 == 0 and w % nl == 0
    nvec = w // nl
    mesh = plsc.VectorSubcoreMesh(core_axis_name="c", subcore_axis_name="s")

    @functools.partial(
        pl.kernel,
        mesh=mesh,
        compiler_params=pltpu.CompilerParams(
            needs_layout_passes=False,
            use_tc_tiling_on_sc=True,
            skip_device_barrier=True,
        ),
        out_type=jax.ShapeDtypeStruct((hw * nbins,), jnp.float32),
        scratch_types=[
            pltpu.VMEM((lrw, w), jnp.int32),
            pltpu.VMEM((nl * nbins,), jnp.float32),
            pltpu.VMEM((nbins,), jnp.float32),
            pltpu.SemaphoreType.DMA,
        ],
    )
    def sc_kernel(lbl_hbm, hist_out, lbl_v, hist_v, part_v, sem_l):
        wid = lax.axis_index("s") * nc + lax.axis_index("c")

        @pl.when(wid < hw)
        def _hist():
            lbl_cp = pltpu.async_copy(
                lbl_hbm.at[pl.ds(wid * lrw, lrw)], lbl_v, sem_l
            )

            def _zero(t, carry):
                for u in range(8):
                    hist_v[pl.ds((t * 8 + u) * nl, nl)] = jnp.zeros(
                        (nl,), jnp.float32
                    )
                return carry

            lax.fori_loop(0, (nl * nbins) // (nl * 8), _zero, 0)
            lbl_cp.wait()

            lane_base = lax.iota(jnp.int32, nl) * nbins
            ones = jnp.ones((nl,), jnp.float32)

            def _scat(r, carry):
                for v in range(nvec):
                    idx = lbl_v[r, pl.ds(v * nl, nl)]
                    plsc.addupdate_scatter(hist_v, [lane_base + idx], ones)
                return carry

            lax.fori_loop(0, lrw, _scat, 0)

            def _red(cchunk, carry):
                def _lane(l, acc):
                    a0, a1 = acc
                    b = 2 * l * nbins + cchunk * nl
                    return (a0 + hist_v[pl.ds(b, nl)],
                            a1 + hist_v[pl.ds(b + nbins, nl)])

                z = jnp.zeros((nl,), jnp.float32)
                a0, a1 = lax.fori_loop(0, nl // 2, _lane, (z, z))
                part_v[pl.ds(cchunk * nl, nl)] = a0 + a1
                return carry

            lax.fori_loop(0, nbins // nl, _red, 0)

            pltpu.sync_copy(part_v, hist_out.at[pl.ds(wid * nbins, nbins)])

    return sc_kernel


def kernel(image, label):
    c, h, w = image.shape
    n = h * w
    nbins = _EMB - c
    mean_c = _channel_means_tc(image)[:, 0]
    hist_parts = _make_hist_sc(h, w, nbins)(label)
    hist = hist_parts.reshape(-1, nbins).sum(axis=0) * (1.0 / n)
    return jnp.concatenate([mean_c, hist])


# speedup vs baseline: 1.0303x; 1.0200x over previous
import functools

import jax
import jax.numpy as jnp
from jax import lax
from jax.experimental import pallas as pl
from jax.experimental.pallas import tpu as pltpu
from jax.experimental.pallas import tpu_sc as plsc

_EMB = 448


def _mean_body(nblk, inv_n, x_ref, o_ref, acc_ref):
    i = pl.program_id(0)

    @pl.when(i == 0)
    def _init():
        acc_ref[...] = jnp.zeros_like(acc_ref)

    x = x_ref[...]
    c, bh, w = x.shape
    p = x[:, 0:8, :]
    for g in range(1, bh // 8):
        p = p + x[:, 8 * g : 8 * (g + 1), :]
    acc_ref[...] += p

    @pl.when(i == nblk - 1)
    def _fin():
        o_ref[...] = acc_ref[...].sum(axis=(1, 2))[:, None] * inv_n


def _channel_means_tc(image):
    c, h, w = image.shape
    bh = 32
    assert h % bh == 0
    nblk = h // bh
    return pl.pallas_call(
        functools.partial(_mean_body, nblk, 1.0 / (h * w)),
        grid=(nblk,),
        in_specs=[pl.BlockSpec((c, bh, w), lambda i: (0, i, 0))],
        out_specs=pl.BlockSpec((c, 1), lambda i: (0, 0)),
        out_shape=jax.ShapeDtypeStruct((c, 1), jnp.float32),
        scratch_shapes=[pltpu.VMEM((c, 8, w), jnp.float32)],
    )(image)


def _make_hist_sc(h, w, nbins):
    info = plsc.get_sparse_core_info()
    nc, ns, nl = info.num_cores, info.num_subcores, info.num_lanes
    nw = nc * ns
    hw = 16
    lrw = h // hw
    assert h % hw == 0 and lrw % 8 == 0 and w % nl == 0
    nvec = w // nl
    mesh = plsc.VectorSubcoreMesh(core_axis_name="c", subcore_axis_name="s")

    @functools.partial(
        pl.kernel,
        mesh=mesh,
        compiler_params=pltpu.CompilerParams(
            needs_layout_passes=False,
            use_tc_tiling_on_sc=True,
            skip_device_barrier=True,
        ),
        out_type=jax.ShapeDtypeStruct((hw * nbins,), jnp.float32),
        scratch_types=[
            pltpu.VMEM((lrw, w), jnp.int32),
            pltpu.VMEM((nl * nbins,), jnp.float32),
            pltpu.VMEM((nbins,), jnp.float32),
            pltpu.SemaphoreType.DMA,
        ],
    )
    def sc_kernel(lbl_hbm, hist_out, lbl_v, hist_v, part_v, sem_l):
        wid = lax.axis_index("s") * nc + lax.axis_index("c")

        @pl.when(wid < hw)
        def _hist():
            lbl_cp = pltpu.async_copy(
                lbl_hbm.at[pl.ds(wid * lrw, lrw)], lbl_v, sem_l
            )

            def _zero(t, carry):
                for u in range(8):
                    hist_v[pl.ds((t * 8 + u) * nl, nl)] = jnp.zeros(
                        (nl,), jnp.float32
                    )
                return carry

            lax.fori_loop(0, (nl * nbins) // (nl * 8), _zero, 0)
            lbl_cp.wait()

            lane_base = lax.iota(jnp.int32, nl) * nbins
            ones = jnp.ones((nl,), jnp.float32)

            def _scat(r, carry):
                for v in range(nvec):
                    idx = lbl_v[r, pl.ds(v * nl, nl)]
                    plsc.addupdate_scatter(hist_v, [lane_base + idx], ones)
                return carry

            lax.fori_loop(0, lrw, _scat, 0)

            def _red(cchunk, carry):
                def _lane(l, acc):
                    a0, a1 = acc
                    b = 2 * l * nbins + cchunk * nl
                    return (a0 + hist_v[pl.ds(b, nl)],
                            a1 + hist_v[pl.ds(b + nbins, nl)])

                z = jnp.zeros((nl,), jnp.float32)
                a0, a1 = lax.fori_loop(0, nl // 2, _lane, (z, z))
                part_v[pl.ds(cchunk * nl, nl)] = a0 + a1
                return carry

            lax.fori_loop(0, nbins // nl, _red, 0)

            pltpu.sync_copy(part_v, hist_out.at[pl.ds(wid * nbins, nbins)])

    return sc_kernel


def _emit_body(nparts, nbins, inv_n, m_ref, h_ref, o_ref):
    hist = h_ref[pl.ds(0, nbins)]
    for k in range(1, nparts):
        hist = hist + h_ref[pl.ds(k * nbins, nbins)]
    o_ref[pl.ds(0, m_ref.shape[0])] = m_ref[...][:, 0]
    o_ref[pl.ds(m_ref.shape[0], nbins)] = hist * inv_n


def _emit(mean2d, hist_parts, nbins, inv_n):
    c = mean2d.shape[0]
    nparts = hist_parts.shape[0] // nbins
    return pl.pallas_call(
        functools.partial(_emit_body, nparts, nbins, inv_n),
        out_shape=jax.ShapeDtypeStruct((c + nbins,), jnp.float32),
    )(mean2d, hist_parts)


def kernel(image, label):
    c, h, w = image.shape
    n = h * w
    nbins = _EMB - c
    mean2d = _channel_means_tc(image)
    hist_parts = _make_hist_sc(h, w, nbins)(label)
    return _emit(mean2d, hist_parts, nbins, 1.0 / n)
